# baseline (speedup 1.0000x reference)
import jax
import jax.numpy as jnp
from jax.experimental import pallas as pl


_NB1 = 2000
_MB = 400


def _project_body(x_ref, w_ref, wl_ref, v_ref, sl_ref):
    k = w_ref.shape[1]
    x0 = x_ref[0]
    x1 = x_ref[1]
    w = w_ref[:]
    wl = wl_ref[:]
    v_ref[:, :k] = jnp.dot(x0, w, preferred_element_type=jnp.float32)
    v_ref[:, k:] = jnp.dot(x1, w, preferred_element_type=jnp.float32)
    sl_ref[0] = jnp.dot(x0, wl, preferred_element_type=jnp.float32)
    sl_ref[1] = jnp.dot(x1, wl, preferred_element_type=jnp.float32)


def _aggregate_body(adj_ref, v_ref, sl_ref, b_ref, out_ref):
    k = b_ref.shape[1]
    acc = jnp.dot(
        adj_ref[:].astype(jnp.bfloat16),
        v_ref[:].astype(jnp.bfloat16),
        preferred_element_type=jnp.float32,
    )
    b = b_ref[:]
    out_ref[0] = jnp.maximum(acc[:, :k] + sl_ref[0] + b, 0.0)
    out_ref[1] = jnp.maximum(acc[:, k:] + sl_ref[1] + b, 0.0)


def kernel(inputs, adj_mat, weight, loop_weight, bias):
    batch, n, f = inputs.shape
    k = weight.shape[1]

    v, sl = pl.pallas_call(
        _project_body,
        grid=(n // _NB1,),
        in_specs=[
            pl.BlockSpec((batch, _NB1, f), lambda i: (0, i, 0)),
            pl.BlockSpec((f, k), lambda i: (0, 0)),
            pl.BlockSpec((f, k), lambda i: (0, 0)),
        ],
        out_specs=[
            pl.BlockSpec((_NB1, batch * k), lambda i: (i, 0)),
            pl.BlockSpec((batch, _NB1, k), lambda i: (0, i, 0)),
        ],
        out_shape=[
            jax.ShapeDtypeStruct((n, batch * k), jnp.float32),
            jax.ShapeDtypeStruct((batch, n, k), jnp.float32),
        ],
    )(inputs, weight, loop_weight)

    out = pl.pallas_call(
        _aggregate_body,
        grid=(n // _MB,),
        in_specs=[
            pl.BlockSpec((_MB, n), lambda i: (i, 0)),
            pl.BlockSpec((n, batch * k), lambda i: (0, 0)),
            pl.BlockSpec((batch, _MB, k), lambda i: (0, i, 0)),
            pl.BlockSpec((1, k), lambda i: (0, 0)),
        ],
        out_specs=pl.BlockSpec((batch, _MB, k), lambda i: (0, i, 0)),
        out_shape=jax.ShapeDtypeStruct((batch, n, k), jnp.float32),
    )(adj_mat, v, sl, bias.reshape(1, k))
    return out


# speedup vs baseline: 1.0409x; 1.0409x over previous
import jax
import jax.numpy as jnp
from jax.experimental import pallas as pl
from jax.experimental.pallas import tpu as pltpu


_MB = 200


def _gconv_body(x_ref, adj_ref, w_ref, wl_ref, b_ref, out_ref, v_ref):
    k = w_ref.shape[1]
    i = pl.program_id(0)

    @pl.when(i == 0)
    def _build_v():
        w = w_ref[:]
        v_ref[:, :k] = jnp.dot(
            x_ref[0], w, preferred_element_type=jnp.float32
        ).astype(jnp.bfloat16)
        v_ref[:, k:] = jnp.dot(
            x_ref[1], w, preferred_element_type=jnp.float32
        ).astype(jnp.bfloat16)

    acc = jnp.dot(
        adj_ref[:].astype(jnp.bfloat16),
        v_ref[:],
        preferred_element_type=jnp.float32,
    )
    x_m = x_ref[:, pl.ds(i * _MB, _MB), :]
    wl = wl_ref[:]
    b = b_ref[:]
    sl0 = jnp.dot(x_m[0], wl, preferred_element_type=jnp.float32)
    sl1 = jnp.dot(x_m[1], wl, preferred_element_type=jnp.float32)
    out_ref[0] = jnp.maximum(acc[:, :k] + sl0 + b, 0.0)
    out_ref[1] = jnp.maximum(acc[:, k:] + sl1 + b, 0.0)


def kernel(inputs, adj_mat, weight, loop_weight, bias):
    batch, n, f = inputs.shape
    k = weight.shape[1]

    return pl.pallas_call(
        _gconv_body,
        grid=(n // _MB,),
        in_specs=[
            pl.BlockSpec((batch, n, f), lambda i: (0, 0, 0)),
            pl.BlockSpec((_MB, n), lambda i: (i, 0)),
            pl.BlockSpec((f, k), lambda i: (0, 0)),
            pl.BlockSpec((f, k), lambda i: (0, 0)),
            pl.BlockSpec((1, k), lambda i: (0, 0)),
        ],
        out_specs=pl.BlockSpec((batch, _MB, k), lambda i: (0, i, 0)),
        out_shape=jax.ShapeDtypeStruct((batch, n, k), jnp.float32),
        scratch_shapes=[pltpu.VMEM((n, batch * k), jnp.bfloat16)],
    )(inputs, adj_mat, weight, loop_weight, bias.reshape(1, k))


# speedup vs baseline: 1.0840x; 1.0414x over previous
import jax
import jax.numpy as jnp
from jax.experimental import pallas as pl
from jax.experimental.pallas import tpu as pltpu


_MB = 200


def _gconv_body(x_ref, adj_ref, w_ref, wl_ref, b_ref, out_ref, v_ref, sl_ref):
    k = w_ref.shape[1]
    i = pl.program_id(0)

    @pl.when(i == 0)
    def _build_panels():
        w = w_ref[:]
        wl = wl_ref[:]
        x0 = x_ref[0]
        x1 = x_ref[1]
        v_ref[:, :k] = jnp.dot(
            x0, w, preferred_element_type=jnp.float32
        ).astype(jnp.bfloat16)
        v_ref[:, k:] = jnp.dot(
            x1, w, preferred_element_type=jnp.float32
        ).astype(jnp.bfloat16)
        sl_ref[:, :k] = jnp.dot(x0, wl, preferred_element_type=jnp.float32)
        sl_ref[:, k:] = jnp.dot(x1, wl, preferred_element_type=jnp.float32)

    acc = jnp.dot(
        adj_ref[:].astype(jnp.bfloat16),
        v_ref[:],
        preferred_element_type=jnp.float32,
    )
    sl = sl_ref[pl.ds(i * _MB, _MB), :]
    out_ref[:] = jnp.maximum(acc + sl + b_ref[:], 0.0)


def kernel(inputs, adj_mat, weight, loop_weight, bias):
    batch, n, f = inputs.shape
    k = weight.shape[1]

    packed = pl.pallas_call(
        _gconv_body,
        grid=(n // _MB,),
        in_specs=[
            pl.BlockSpec((batch, n, f), lambda i: (0, 0, 0)),
            pl.BlockSpec((_MB, n), lambda i: (i, 0)),
            pl.BlockSpec((f, k), lambda i: (0, 0)),
            pl.BlockSpec((f, k), lambda i: (0, 0)),
            pl.BlockSpec((1, batch * k), lambda i: (0, 0)),
        ],
        out_specs=pl.BlockSpec((_MB, batch * k), lambda i: (i, 0)),
        out_shape=jax.ShapeDtypeStruct((n, batch * k), jnp.float32),
        scratch_shapes=[
            pltpu.VMEM((n, batch * k), jnp.bfloat16),
            pltpu.VMEM((n, batch * k), jnp.float32),
        ],
    )(
        inputs,
        adj_mat,
        weight,
        loop_weight,
        jnp.tile(bias, batch).reshape(1, batch * k),
    )
    return jnp.transpose(packed.reshape(n, batch, k), (1, 0, 2))
